# baseline (speedup 1.0000x reference)
import jax
import jax.numpy as jnp
from jax.experimental import pallas as pl
from jax.experimental.pallas import tpu as pltpu

_N = 2048
_QT = 512
_K = 20
_CH = 64
_NC = _N // _CH
_L = 5


def _knn_cov_kernel(xb_ref, rowsT_ref, out_ref, work_ref, st_ref,
                    V_ref, C_ref):
    xb = xb_ref[...]
    rowsT = rowsT_ref[...]
    xb16 = xb.astype(jnp.bfloat16)
    rowsT16 = rowsT.astype(jnp.bfloat16)
    prod = jnp.dot(xb16, rowsT16, preferred_element_type=jnp.float32)
    sqa = jnp.sum(xb * xb, axis=1, keepdims=True)
    sqr = jnp.sum(rowsT * rowsT, axis=0, keepdims=True)
    d2 = jnp.maximum(sqa + sqr - 2.0 * prod, 0.0)
    dd = jnp.sqrt(d2)

    kf = jnp.float32(_K)

    for ci in range(_NC):
        w = dd[ci * _CH:(ci + 1) * _CH, :]
        vrows = []
        crows = []
        for l in range(_L):
            m = jnp.min(w, axis=0, keepdims=True)
            eqw = w == m
            cntl = jnp.sum(eqw.astype(jnp.float32), axis=0, keepdims=True)
            cntl = jnp.where(m == jnp.inf, 0.0, cntl)
            vrows.append(m)
            crows.append(cntl)
            w = jnp.where(eqw, jnp.inf, w)
        V_ref[ci * _L:(ci + 1) * _L, :] = jnp.concatenate(vrows, axis=0)
        C_ref[ci * _L:(ci + 1) * _L, :] = jnp.concatenate(crows, axis=0)

    mlast = jnp.concatenate(
        [V_ref[ci * _L + _L - 1:ci * _L + _L, :] for ci in range(_NC)], axis=0)

    st_ref[0:1, :] = jnp.full((1, _QT), jnp.inf, jnp.float32)
    st_ref[1:4, :] = jnp.zeros((3, _QT), jnp.float32)

    def pop_body(_, c):
        V = V_ref[...]
        m = jnp.min(V, axis=0, keepdims=True)
        eqv = V == m
        ceq = jnp.sum(jnp.where(eqv, C_ref[...], 0.0), axis=0, keepdims=True)
        c_prev = st_ref[2:3, :]
        found = st_ref[3:4, :]
        c_le = c_prev + ceq
        newly = jnp.logical_and(found == 0.0, c_le >= kf)
        st_ref[0:1, :] = jnp.where(newly, m, st_ref[0:1, :])
        st_ref[1:2, :] = jnp.where(newly, c_prev, st_ref[1:2, :])
        st_ref[2:3, :] = c_le
        st_ref[3:4, :] = jnp.where(newly, 1.0, found)
        V_ref[...] = jnp.where(eqv, jnp.inf, V)
        return c

    jax.lax.fori_loop(0, _K, pop_body, 0)

    m6min = jnp.min(mlast, axis=0, keepdims=True)
    unsafe = jnp.any(st_ref[0:1, :] > m6min)

    @pl.when(unsafe)
    def _fallback():
        work_ref[...] = dd
        st_ref[0:1, :] = jnp.full((1, _QT), jnp.inf, jnp.float32)
        st_ref[1:4, :] = jnp.zeros((3, _QT), jnp.float32)

        def body(_, c):
            work = work_ref[...]
            m = jnp.min(work, axis=0, keepdims=True)
            eqw = work == m
            ceq = jnp.sum(eqw.astype(jnp.float32), axis=0, keepdims=True)
            c_prev = st_ref[2:3, :]
            found = st_ref[3:4, :]
            c_le = c_prev + ceq
            newly = jnp.logical_and(found == 0.0, c_le >= kf)
            st_ref[0:1, :] = jnp.where(newly, m, st_ref[0:1, :])
            st_ref[1:2, :] = jnp.where(newly, c_prev, st_ref[1:2, :])
            st_ref[2:3, :] = c_le
            st_ref[3:4, :] = jnp.where(newly, 1.0, found)
            work_ref[...] = jnp.where(eqw, jnp.inf, work)
            return c

        jax.lax.fori_loop(0, _K, body, 0)

    v20 = st_ref[0:1, :]
    nless = st_ref[1:2, :]

    mask0 = (dd <= v20).astype(jnp.float32)
    cnt20 = jnp.sum(mask0, axis=0, keepdims=True)
    dirty = jnp.any(cnt20 > kf)

    @pl.when(jnp.logical_not(dirty))
    def _clean():
        work_ref[...] = mask0

    @pl.when(dirty)
    def _tiebreak():
        eq = (dd == v20).astype(jnp.float32)
        pre = eq
        sh = 1
        while sh < _N:
            pre = pre + jnp.concatenate(
                [jnp.zeros((sh, _QT), jnp.float32), pre[:_N - sh, :]], axis=0)
            sh *= 2
        sel = jnp.logical_or(dd < v20,
                             jnp.logical_and(eq > 0.0, pre <= (kf - nless)))
        work_ref[...] = sel.astype(jnp.float32)

    maskf = work_ref[...]
    sel = maskf > 0.0
    cnt = jnp.sum(maskf, axis=0, keepdims=True)
    inv = 1.0 / cnt
    xs = [xb[:, a:a + 1] for a in range(3)]
    s1 = [jnp.sum(jnp.where(sel, xs[a], 0.0), axis=0, keepdims=True)
          for a in range(3)]
    cen = [s1[a] * inv for a in range(3)]
    D = [jnp.where(sel, (xs[a] - cen[a]).astype(jnp.bfloat16)
                   .astype(jnp.float32), 0.0) for a in range(3)]
    covs = []
    for (i, j) in ((0, 0), (0, 1), (0, 2), (1, 1), (1, 2), (2, 2)):
        covs.append(jnp.sum(D[i] * D[j], axis=0, keepdims=True))
    zero = jnp.zeros((1, _QT), jnp.float32)
    out_ref[...] = jnp.concatenate(
        covs + cen + [cnt] + [zero] * 6, axis=0)


def _rot(p, q, a, v2):
    app = a[(p, p)]
    aqq = a[(q, q)]
    apq = a[(p, q)]
    tau = (aqq - app) / (2.0 * apq)
    sgn = jnp.where(tau >= 0.0, 1.0, -1.0)
    t = sgn / (jnp.abs(tau) + jnp.sqrt(1.0 + tau * tau))
    t = jnp.where(apq == 0.0, 0.0, t)
    c = jax.lax.rsqrt(1.0 + t * t)
    s = t * c
    r = 3 - p - q
    arp = a[(min(r, p), max(r, p))]
    arq = a[(min(r, q), max(r, q))]
    a2 = dict(a)
    a2[(p, p)] = c * c * app - 2.0 * c * s * apq + s * s * aqq
    a2[(q, q)] = s * s * app + 2.0 * c * s * apq + c * c * aqq
    a2[(p, q)] = jnp.zeros_like(apq)
    a2[(min(r, p), max(r, p))] = c * arp - s * arq
    a2[(min(r, q), max(r, q))] = s * arp + c * arq
    vp, vq = v2[p], v2[q]
    v2 = list(v2)
    v2[p] = c * vp - s * vq
    v2[q] = s * vp + c * vq
    return a2, v2


def _eig_feat_kernel(s_ref, x_ref, f_ref):
    a = {(0, 0): s_ref[0], (0, 1): s_ref[1], (0, 2): s_ref[2],
         (1, 1): s_ref[3], (1, 2): s_ref[4], (2, 2): s_ref[5]}
    v2 = [jnp.zeros_like(s_ref[0]), jnp.zeros_like(s_ref[0]),
          jnp.ones_like(s_ref[0])]
    for _ in range(4):
        for (p, q) in ((0, 2), (1, 2), (0, 1)):
            a, v2 = _rot(p, q, a, v2)
    w = [jnp.maximum(a[(0, 0)], 0.0), jnp.maximum(a[(1, 1)], 0.0),
         jnp.maximum(a[(2, 2)], 0.0)]

    def rank(i):
        r = jnp.zeros_like(w[0])
        for j in range(3):
            if j == i:
                continue
            gt = w[j] > w[i]
            tie = jnp.logical_and(w[j] == w[i], j < i)
            r = r + jnp.where(jnp.logical_or(gt, tie), 1.0, 0.0)
        return r

    ranks = [rank(i) for i in range(3)]
    for k in range(3):
        nk = jnp.zeros_like(w[0])
        for i in range(3):
            nk = nk + jnp.where(ranks[i] == float(k), v2[i], 0.0)
        f_ref[3 + k] = nk
    for k in range(3):
        f_ref[k] = x_ref[k]
        f_ref[6 + k] = s_ref[6 + k]
    zero = jnp.zeros_like(s_ref[0])
    for k in range(9, 16):
        f_ref[k] = zero


def _mlp_kernel(f_ref, w1_ref, b1_ref, w2_ref, b2_ref, w3_ref, b3_ref, o_ref):
    f = f_ref[...]
    h = jnp.dot(w1_ref[...].astype(jnp.bfloat16), f.astype(jnp.bfloat16),
                preferred_element_type=jnp.float32)
    h = jnp.maximum(h + b1_ref[...], 0.0)
    h = jnp.dot(w2_ref[...].astype(jnp.bfloat16), h.astype(jnp.bfloat16),
                preferred_element_type=jnp.float32)
    h = jnp.maximum(h + b2_ref[...], 0.0)
    h = jnp.dot(w3_ref[...].astype(jnp.bfloat16), h.astype(jnp.bfloat16),
                preferred_element_type=jnp.float32)
    o_ref[...] = h + b3_ref[...]


@jax.jit
def kernel(point_cloud, vis_mask, W1, b1, W2, b2, W3, b3):
    B, N, _ = point_cloud.shape
    pts = point_cloud * vis_mask[:, :, None].astype(point_cloud.dtype)
    ptsT = jnp.transpose(pts, (0, 2, 1))

    stats = pl.pallas_call(
        _knn_cov_kernel,
        grid=(B, N // _QT),
        in_specs=[
            pl.BlockSpec((None, N, 3), lambda b, i: (b, 0, 0)),
            pl.BlockSpec((None, 3, _QT), lambda b, i: (b, 0, i)),
        ],
        out_specs=pl.BlockSpec((None, 16, _QT), lambda b, i: (b, 0, i)),
        out_shape=jax.ShapeDtypeStruct((B, 16, N), jnp.float32),
        scratch_shapes=[pltpu.VMEM((N, _QT), jnp.float32),
                        pltpu.VMEM((8, _QT), jnp.float32),
                        pltpu.VMEM((_NC * _L, _QT), jnp.float32),
                        pltpu.VMEM((_NC * _L, _QT), jnp.float32)],
    )(pts, ptsT)

    P = B * N // 128
    s_planes = jnp.transpose(stats, (1, 0, 2)).reshape(16, P, 128)
    x_planes = jnp.transpose(ptsT, (1, 0, 2)).reshape(3, P, 128)

    feats = pl.pallas_call(
        _eig_feat_kernel,
        in_specs=[pl.BlockSpec((16, P, 128), lambda: (0, 0, 0)),
                  pl.BlockSpec((3, P, 128), lambda: (0, 0, 0))],
        out_specs=pl.BlockSpec((16, P, 128), lambda: (0, 0, 0)),
        out_shape=jax.ShapeDtypeStruct((16, P, 128), jnp.float32),
    )(s_planes, x_planes)

    fmat = jnp.transpose(feats.reshape(16, B, N), (1, 0, 2))
    w1p = jnp.pad(W1, ((0, 0), (0, 16 - W1.shape[1])))
    out = pl.pallas_call(
        _mlp_kernel,
        grid=(B,),
        in_specs=[
            pl.BlockSpec((None, 16, N), lambda b: (b, 0, 0)),
            pl.BlockSpec((64, 16), lambda b: (0, 0)),
            pl.BlockSpec((64, 1), lambda b: (0, 0)),
            pl.BlockSpec((128, 64), lambda b: (0, 0)),
            pl.BlockSpec((128, 1), lambda b: (0, 0)),
            pl.BlockSpec((256, 128), lambda b: (0, 0)),
            pl.BlockSpec((256, 1), lambda b: (0, 0)),
        ],
        out_specs=pl.BlockSpec((None, 256, N), lambda b: (b, 0, 0)),
        out_shape=jax.ShapeDtypeStruct((B, 256, N), jnp.float32),
    )(fmat, w1p, b1[:, None], W2, b2[:, None], W3, b3[:, None])
    return out


# speedup vs baseline: 1.7902x; 1.7902x over previous
import jax
import jax.numpy as jnp
from jax.experimental import pallas as pl
from jax.experimental.pallas import tpu as pltpu

_N = 2048
_QT = 512
_K = 20
_CH = 64
_NC = _N // _CH
_L = 6


def _knn_cov_kernel(xb_ref, rowsT_ref, out_ref, work_ref, st_ref,
                    V_ref, C_ref):
    xb = xb_ref[...]
    rowsT = rowsT_ref[...]
    xb16 = xb.astype(jnp.bfloat16)
    rowsT16 = rowsT.astype(jnp.bfloat16)
    prod = jnp.dot(xb16, rowsT16, preferred_element_type=jnp.float32)
    sqa = jnp.sum(xb * xb, axis=1, keepdims=True)
    sqr = jnp.sum(rowsT * rowsT, axis=0, keepdims=True)
    d2 = jnp.maximum(sqa + sqr - 2.0 * prod, 0.0)
    dd = jnp.sqrt(d2)

    kf = jnp.float32(_K)

    for ci in range(_NC):
        w = dd[ci * _CH:(ci + 1) * _CH, :]
        vrows = []
        crows = []
        for l in range(_L):
            m = jnp.min(w, axis=0, keepdims=True)
            eqw = w == m
            cntl = jnp.sum(eqw.astype(jnp.float32), axis=0, keepdims=True)
            cntl = jnp.where(m == jnp.inf, 0.0, cntl)
            vrows.append(m)
            crows.append(cntl)
            w = jnp.where(eqw, jnp.inf, w)
        V_ref[ci * _L:(ci + 1) * _L, :] = jnp.concatenate(vrows, axis=0)
        C_ref[ci * _L:(ci + 1) * _L, :] = jnp.concatenate(crows, axis=0)

    mlast = jnp.concatenate(
        [V_ref[ci * _L + _L - 1:ci * _L + _L, :] for ci in range(_NC)], axis=0)

    st_ref[0:1, :] = jnp.full((1, _QT), jnp.inf, jnp.float32)
    st_ref[1:4, :] = jnp.zeros((3, _QT), jnp.float32)

    def pop_body(_, c):
        V = V_ref[...]
        m = jnp.min(V, axis=0, keepdims=True)
        eqv = V == m
        ceq = jnp.sum(jnp.where(eqv, C_ref[...], 0.0), axis=0, keepdims=True)
        c_prev = st_ref[2:3, :]
        found = st_ref[3:4, :]
        c_le = c_prev + ceq
        newly = jnp.logical_and(found == 0.0, c_le >= kf)
        st_ref[0:1, :] = jnp.where(newly, m, st_ref[0:1, :])
        st_ref[1:2, :] = jnp.where(newly, c_prev, st_ref[1:2, :])
        st_ref[2:3, :] = c_le
        st_ref[3:4, :] = jnp.where(newly, 1.0, found)
        V_ref[...] = jnp.where(eqv, jnp.inf, V)
        return c

    jax.lax.fori_loop(0, _K, pop_body, 0)

    m6min = jnp.min(mlast, axis=0, keepdims=True)
    unsafe = jnp.any(st_ref[0:1, :] > m6min)

    @pl.when(unsafe)
    def _fallback():
        work_ref[...] = dd
        st_ref[0:1, :] = jnp.full((1, _QT), jnp.inf, jnp.float32)
        st_ref[1:4, :] = jnp.zeros((3, _QT), jnp.float32)

        def body(_, c):
            work = work_ref[...]
            m = jnp.min(work, axis=0, keepdims=True)
            eqw = work == m
            ceq = jnp.sum(eqw.astype(jnp.float32), axis=0, keepdims=True)
            c_prev = st_ref[2:3, :]
            found = st_ref[3:4, :]
            c_le = c_prev + ceq
            newly = jnp.logical_and(found == 0.0, c_le >= kf)
            st_ref[0:1, :] = jnp.where(newly, m, st_ref[0:1, :])
            st_ref[1:2, :] = jnp.where(newly, c_prev, st_ref[1:2, :])
            st_ref[2:3, :] = c_le
            st_ref[3:4, :] = jnp.where(newly, 1.0, found)
            work_ref[...] = jnp.where(eqw, jnp.inf, work)
            return c

        jax.lax.fori_loop(0, _K, body, 0)

    v20 = st_ref[0:1, :]
    nless = st_ref[1:2, :]

    mask0 = (dd <= v20).astype(jnp.float32)
    cnt20 = jnp.sum(mask0, axis=0, keepdims=True)
    dirty = jnp.any(cnt20 > kf)

    @pl.when(jnp.logical_not(dirty))
    def _clean():
        work_ref[...] = mask0

    @pl.when(dirty)
    def _tiebreak():
        eq = (dd == v20).astype(jnp.float32)
        pre = eq
        sh = 1
        while sh < _N:
            pre = pre + jnp.concatenate(
                [jnp.zeros((sh, _QT), jnp.float32), pre[:_N - sh, :]], axis=0)
            sh *= 2
        sel = jnp.logical_or(dd < v20,
                             jnp.logical_and(eq > 0.0, pre <= (kf - nless)))
        work_ref[...] = sel.astype(jnp.float32)

    maskf = work_ref[...]
    sel = maskf > 0.0
    inv = jnp.float32(1.0 / _K)
    xs = [xb[:, a:a + 1] for a in range(3)]
    s1 = [jnp.sum(jnp.where(sel, xs[a], 0.0), axis=0, keepdims=True)
          for a in range(3)]
    cen = [s1[a] * inv for a in range(3)]
    D = [jnp.where(sel, (xs[a] - cen[a]).astype(jnp.bfloat16)
                   .astype(jnp.float32), 0.0) for a in range(3)]
    covs = []
    for (i, j) in ((0, 0), (0, 1), (0, 2), (1, 1), (1, 2), (2, 2)):
        covs.append(jnp.sum(D[i] * D[j], axis=0, keepdims=True))
    zero = jnp.zeros((1, _QT), jnp.float32)
    out_ref[...] = jnp.concatenate(
        covs + cen + [zero] * 7, axis=0)


def _rot(p, q, a, v2):
    app = a[(p, p)]
    aqq = a[(q, q)]
    apq = a[(p, q)]
    tau = (aqq - app) / (2.0 * apq)
    sgn = jnp.where(tau >= 0.0, 1.0, -1.0)
    t = sgn / (jnp.abs(tau) + jnp.sqrt(1.0 + tau * tau))
    t = jnp.where(apq == 0.0, 0.0, t)
    c = jax.lax.rsqrt(1.0 + t * t)
    s = t * c
    r = 3 - p - q
    arp = a[(min(r, p), max(r, p))]
    arq = a[(min(r, q), max(r, q))]
    a2 = dict(a)
    a2[(p, p)] = c * c * app - 2.0 * c * s * apq + s * s * aqq
    a2[(q, q)] = s * s * app + 2.0 * c * s * apq + c * c * aqq
    a2[(p, q)] = jnp.zeros_like(apq)
    a2[(min(r, p), max(r, p))] = c * arp - s * arq
    a2[(min(r, q), max(r, q))] = s * arp + c * arq
    vp, vq = v2[p], v2[q]
    v2 = list(v2)
    v2[p] = c * vp - s * vq
    v2[q] = s * vp + c * vq
    return a2, v2


def _eig_feat_kernel(s_ref, x_ref, f_ref):
    a = {(0, 0): s_ref[0], (0, 1): s_ref[1], (0, 2): s_ref[2],
         (1, 1): s_ref[3], (1, 2): s_ref[4], (2, 2): s_ref[5]}
    v2 = [jnp.zeros_like(s_ref[0]), jnp.zeros_like(s_ref[0]),
          jnp.ones_like(s_ref[0])]
    for _ in range(4):
        for (p, q) in ((0, 2), (1, 2), (0, 1)):
            a, v2 = _rot(p, q, a, v2)
    w = [jnp.maximum(a[(0, 0)], 0.0), jnp.maximum(a[(1, 1)], 0.0),
         jnp.maximum(a[(2, 2)], 0.0)]

    def rank(i):
        r = jnp.zeros_like(w[0])
        for j in range(3):
            if j == i:
                continue
            gt = w[j] > w[i]
            tie = jnp.logical_and(w[j] == w[i], j < i)
            r = r + jnp.where(jnp.logical_or(gt, tie), 1.0, 0.0)
        return r

    ranks = [rank(i) for i in range(3)]
    for k in range(3):
        nk = jnp.zeros_like(w[0])
        for i in range(3):
            nk = nk + jnp.where(ranks[i] == float(k), v2[i], 0.0)
        f_ref[3 + k] = nk
    for k in range(3):
        f_ref[k] = x_ref[k]
        f_ref[6 + k] = s_ref[6 + k]
    zero = jnp.zeros_like(s_ref[0])
    for k in range(9, 16):
        f_ref[k] = zero


def _mlp_kernel(f_ref, w1_ref, b1_ref, w2_ref, b2_ref, w3_ref, b3_ref, o_ref):
    f = f_ref[...]
    h = jnp.dot(w1_ref[...].astype(jnp.bfloat16), f.astype(jnp.bfloat16),
                preferred_element_type=jnp.float32)
    h = jnp.maximum(h + b1_ref[...], 0.0)
    h = jnp.dot(w2_ref[...].astype(jnp.bfloat16), h.astype(jnp.bfloat16),
                preferred_element_type=jnp.float32)
    h = jnp.maximum(h + b2_ref[...], 0.0)
    h = jnp.dot(w3_ref[...].astype(jnp.bfloat16), h.astype(jnp.bfloat16),
                preferred_element_type=jnp.float32)
    o_ref[...] = h + b3_ref[...]


@jax.jit
def kernel(point_cloud, vis_mask, W1, b1, W2, b2, W3, b3):
    B, N, _ = point_cloud.shape
    pts = point_cloud * vis_mask[:, :, None].astype(point_cloud.dtype)
    ptsT = jnp.transpose(pts, (0, 2, 1))

    stats = pl.pallas_call(
        _knn_cov_kernel,
        grid=(B, N // _QT),
        in_specs=[
            pl.BlockSpec((None, N, 3), lambda b, i: (b, 0, 0)),
            pl.BlockSpec((None, 3, _QT), lambda b, i: (b, 0, i)),
        ],
        out_specs=pl.BlockSpec((None, 16, _QT), lambda b, i: (b, 0, i)),
        out_shape=jax.ShapeDtypeStruct((B, 16, N), jnp.float32),
        scratch_shapes=[pltpu.VMEM((N, _QT), jnp.float32),
                        pltpu.VMEM((8, _QT), jnp.float32),
                        pltpu.VMEM((_NC * _L, _QT), jnp.float32),
                        pltpu.VMEM((_NC * _L, _QT), jnp.float32)],
    )(pts, ptsT)

    P = B * N // 128
    s_planes = jnp.transpose(stats, (1, 0, 2)).reshape(16, P, 128)
    x_planes = jnp.transpose(ptsT, (1, 0, 2)).reshape(3, P, 128)

    feats = pl.pallas_call(
        _eig_feat_kernel,
        in_specs=[pl.BlockSpec((16, P, 128), lambda: (0, 0, 0)),
                  pl.BlockSpec((3, P, 128), lambda: (0, 0, 0))],
        out_specs=pl.BlockSpec((16, P, 128), lambda: (0, 0, 0)),
        out_shape=jax.ShapeDtypeStruct((16, P, 128), jnp.float32),
    )(s_planes, x_planes)

    fmat = jnp.transpose(feats.reshape(16, B, N), (1, 0, 2))
    w1p = jnp.pad(W1, ((0, 0), (0, 16 - W1.shape[1])))
    out = pl.pallas_call(
        _mlp_kernel,
        grid=(B,),
        in_specs=[
            pl.BlockSpec((None, 16, N), lambda b: (b, 0, 0)),
            pl.BlockSpec((64, 16), lambda b: (0, 0)),
            pl.BlockSpec((64, 1), lambda b: (0, 0)),
            pl.BlockSpec((128, 64), lambda b: (0, 0)),
            pl.BlockSpec((128, 1), lambda b: (0, 0)),
            pl.BlockSpec((256, 128), lambda b: (0, 0)),
            pl.BlockSpec((256, 1), lambda b: (0, 0)),
        ],
        out_specs=pl.BlockSpec((None, 256, N), lambda b: (b, 0, 0)),
        out_shape=jax.ShapeDtypeStruct((B, 256, N), jnp.float32),
    )(fmat, w1p, b1[:, None], W2, b2[:, None], W3, b3[:, None])
    return out


# speedup vs baseline: 1.8066x; 1.0091x over previous
import jax
import jax.numpy as jnp
from jax.experimental import pallas as pl
from jax.experimental.pallas import tpu as pltpu

_N = 2048
_QT = 256
_K = 20
_CH = 64
_NC = _N // _CH
_L = 6


def _knn_cov_kernel(xb_ref, rowsT_ref, out_ref, work_ref, st_ref,
                    V_ref, C_ref):
    xb = xb_ref[...]
    rowsT = rowsT_ref[...]
    xb16 = xb.astype(jnp.bfloat16)
    rowsT16 = rowsT.astype(jnp.bfloat16)
    prod = jnp.dot(xb16, rowsT16, preferred_element_type=jnp.float32)
    sqa = jnp.sum(xb * xb, axis=1, keepdims=True)
    sqr = jnp.sum(rowsT * rowsT, axis=0, keepdims=True)
    d2 = jnp.maximum(sqa + sqr - 2.0 * prod, 0.0)
    dd = jnp.sqrt(d2)

    kf = jnp.float32(_K)

    for ci in range(_NC):
        w = dd[ci * _CH:(ci + 1) * _CH, :]
        vrows = []
        crows = []
        for l in range(_L):
            m = jnp.min(w, axis=0, keepdims=True)
            eqw = w == m
            cntl = jnp.sum(eqw.astype(jnp.float32), axis=0, keepdims=True)
            cntl = jnp.where(m == jnp.inf, 0.0, cntl)
            vrows.append(m)
            crows.append(cntl)
            w = jnp.where(eqw, jnp.inf, w)
        V_ref[ci * _L:(ci + 1) * _L, :] = jnp.concatenate(vrows, axis=0)
        C_ref[ci * _L:(ci + 1) * _L, :] = jnp.concatenate(crows, axis=0)

    mlast = jnp.concatenate(
        [V_ref[ci * _L + _L - 1:ci * _L + _L, :] for ci in range(_NC)], axis=0)

    st_ref[0:1, :] = jnp.full((1, _QT), jnp.inf, jnp.float32)
    st_ref[1:4, :] = jnp.zeros((3, _QT), jnp.float32)

    def pop_body(_, c):
        V = V_ref[...]
        m = jnp.min(V, axis=0, keepdims=True)
        eqv = V == m
        ceq = jnp.sum(jnp.where(eqv, C_ref[...], 0.0), axis=0, keepdims=True)
        c_prev = st_ref[2:3, :]
        found = st_ref[3:4, :]
        c_le = c_prev + ceq
        newly = jnp.logical_and(found == 0.0, c_le >= kf)
        st_ref[0:1, :] = jnp.where(newly, m, st_ref[0:1, :])
        st_ref[1:2, :] = jnp.where(newly, c_prev, st_ref[1:2, :])
        st_ref[2:3, :] = c_le
        st_ref[3:4, :] = jnp.where(newly, 1.0, found)
        V_ref[...] = jnp.where(eqv, jnp.inf, V)
        return c

    jax.lax.fori_loop(0, _K, pop_body, 0)

    m6min = jnp.min(mlast, axis=0, keepdims=True)
    unsafe = jnp.any(st_ref[0:1, :] > m6min)

    @pl.when(unsafe)
    def _fallback():
        work_ref[...] = dd
        st_ref[0:1, :] = jnp.full((1, _QT), jnp.inf, jnp.float32)
        st_ref[1:4, :] = jnp.zeros((3, _QT), jnp.float32)

        def body(_, c):
            work = work_ref[...]
            m = jnp.min(work, axis=0, keepdims=True)
            eqw = work == m
            ceq = jnp.sum(eqw.astype(jnp.float32), axis=0, keepdims=True)
            c_prev = st_ref[2:3, :]
            found = st_ref[3:4, :]
            c_le = c_prev + ceq
            newly = jnp.logical_and(found == 0.0, c_le >= kf)
            st_ref[0:1, :] = jnp.where(newly, m, st_ref[0:1, :])
            st_ref[1:2, :] = jnp.where(newly, c_prev, st_ref[1:2, :])
            st_ref[2:3, :] = c_le
            st_ref[3:4, :] = jnp.where(newly, 1.0, found)
            work_ref[...] = jnp.where(eqw, jnp.inf, work)
            return c

        jax.lax.fori_loop(0, _K, body, 0)

    v20 = st_ref[0:1, :]
    nless = st_ref[1:2, :]

    mask0 = (dd <= v20).astype(jnp.float32)
    cnt20 = jnp.sum(mask0, axis=0, keepdims=True)
    dirty = jnp.any(cnt20 > kf)

    @pl.when(jnp.logical_not(dirty))
    def _clean():
        work_ref[...] = mask0

    @pl.when(dirty)
    def _tiebreak():
        eq = (dd == v20).astype(jnp.float32)
        pre = eq
        sh = 1
        while sh < _N:
            pre = pre + jnp.concatenate(
                [jnp.zeros((sh, _QT), jnp.float32), pre[:_N - sh, :]], axis=0)
            sh *= 2
        sel = jnp.logical_or(dd < v20,
                             jnp.logical_and(eq > 0.0, pre <= (kf - nless)))
        work_ref[...] = sel.astype(jnp.float32)

    maskf = work_ref[...]
    sel = maskf > 0.0
    inv = jnp.float32(1.0 / _K)
    xs = [xb[:, a:a + 1] for a in range(3)]
    s1 = [jnp.sum(jnp.where(sel, xs[a], 0.0), axis=0, keepdims=True)
          for a in range(3)]
    cen = [s1[a] * inv for a in range(3)]
    D = [jnp.where(sel, (xs[a] - cen[a]).astype(jnp.bfloat16)
                   .astype(jnp.float32), 0.0) for a in range(3)]
    covs = []
    for (i, j) in ((0, 0), (0, 1), (0, 2), (1, 1), (1, 2), (2, 2)):
        covs.append(jnp.sum(D[i] * D[j], axis=0, keepdims=True))
    zero = jnp.zeros((1, _QT), jnp.float32)
    out_ref[...] = jnp.concatenate(
        covs + cen + [zero] * 7, axis=0)


def _rot(p, q, a, v2):
    app = a[(p, p)]
    aqq = a[(q, q)]
    apq = a[(p, q)]
    tau = (aqq - app) / (2.0 * apq)
    sgn = jnp.where(tau >= 0.0, 1.0, -1.0)
    t = sgn / (jnp.abs(tau) + jnp.sqrt(1.0 + tau * tau))
    t = jnp.where(apq == 0.0, 0.0, t)
    c = jax.lax.rsqrt(1.0 + t * t)
    s = t * c
    r = 3 - p - q
    arp = a[(min(r, p), max(r, p))]
    arq = a[(min(r, q), max(r, q))]
    a2 = dict(a)
    a2[(p, p)] = c * c * app - 2.0 * c * s * apq + s * s * aqq
    a2[(q, q)] = s * s * app + 2.0 * c * s * apq + c * c * aqq
    a2[(p, q)] = jnp.zeros_like(apq)
    a2[(min(r, p), max(r, p))] = c * arp - s * arq
    a2[(min(r, q), max(r, q))] = s * arp + c * arq
    vp, vq = v2[p], v2[q]
    v2 = list(v2)
    v2[p] = c * vp - s * vq
    v2[q] = s * vp + c * vq
    return a2, v2


def _eig_feat_kernel(s_ref, x_ref, f_ref):
    a = {(0, 0): s_ref[0], (0, 1): s_ref[1], (0, 2): s_ref[2],
         (1, 1): s_ref[3], (1, 2): s_ref[4], (2, 2): s_ref[5]}
    v2 = [jnp.zeros_like(s_ref[0]), jnp.zeros_like(s_ref[0]),
          jnp.ones_like(s_ref[0])]
    for _ in range(4):
        for (p, q) in ((0, 2), (1, 2), (0, 1)):
            a, v2 = _rot(p, q, a, v2)
    w = [jnp.maximum(a[(0, 0)], 0.0), jnp.maximum(a[(1, 1)], 0.0),
         jnp.maximum(a[(2, 2)], 0.0)]

    def rank(i):
        r = jnp.zeros_like(w[0])
        for j in range(3):
            if j == i:
                continue
            gt = w[j] > w[i]
            tie = jnp.logical_and(w[j] == w[i], j < i)
            r = r + jnp.where(jnp.logical_or(gt, tie), 1.0, 0.0)
        return r

    ranks = [rank(i) for i in range(3)]
    for k in range(3):
        nk = jnp.zeros_like(w[0])
        for i in range(3):
            nk = nk + jnp.where(ranks[i] == float(k), v2[i], 0.0)
        f_ref[3 + k] = nk
    for k in range(3):
        f_ref[k] = x_ref[k]
        f_ref[6 + k] = s_ref[6 + k]
    zero = jnp.zeros_like(s_ref[0])
    for k in range(9, 16):
        f_ref[k] = zero


def _mlp_kernel(f_ref, w1_ref, b1_ref, w2_ref, b2_ref, w3_ref, b3_ref, o_ref):
    f = f_ref[...]
    h = jnp.dot(w1_ref[...].astype(jnp.bfloat16), f.astype(jnp.bfloat16),
                preferred_element_type=jnp.float32)
    h = jnp.maximum(h + b1_ref[...], 0.0)
    h = jnp.dot(w2_ref[...].astype(jnp.bfloat16), h.astype(jnp.bfloat16),
                preferred_element_type=jnp.float32)
    h = jnp.maximum(h + b2_ref[...], 0.0)
    h = jnp.dot(w3_ref[...].astype(jnp.bfloat16), h.astype(jnp.bfloat16),
                preferred_element_type=jnp.float32)
    o_ref[...] = h + b3_ref[...]


@jax.jit
def kernel(point_cloud, vis_mask, W1, b1, W2, b2, W3, b3):
    B, N, _ = point_cloud.shape
    pts = point_cloud * vis_mask[:, :, None].astype(point_cloud.dtype)
    ptsT = jnp.transpose(pts, (0, 2, 1))

    stats = pl.pallas_call(
        _knn_cov_kernel,
        grid=(B, N // _QT),
        in_specs=[
            pl.BlockSpec((None, N, 3), lambda b, i: (b, 0, 0)),
            pl.BlockSpec((None, 3, _QT), lambda b, i: (b, 0, i)),
        ],
        out_specs=pl.BlockSpec((None, 16, _QT), lambda b, i: (b, 0, i)),
        out_shape=jax.ShapeDtypeStruct((B, 16, N), jnp.float32),
        scratch_shapes=[pltpu.VMEM((N, _QT), jnp.float32),
                        pltpu.VMEM((8, _QT), jnp.float32),
                        pltpu.VMEM((_NC * _L, _QT), jnp.float32),
                        pltpu.VMEM((_NC * _L, _QT), jnp.float32)],
    )(pts, ptsT)

    P = B * N // 128
    s_planes = jnp.transpose(stats, (1, 0, 2)).reshape(16, P, 128)
    x_planes = jnp.transpose(ptsT, (1, 0, 2)).reshape(3, P, 128)

    feats = pl.pallas_call(
        _eig_feat_kernel,
        in_specs=[pl.BlockSpec((16, P, 128), lambda: (0, 0, 0)),
                  pl.BlockSpec((3, P, 128), lambda: (0, 0, 0))],
        out_specs=pl.BlockSpec((16, P, 128), lambda: (0, 0, 0)),
        out_shape=jax.ShapeDtypeStruct((16, P, 128), jnp.float32),
    )(s_planes, x_planes)

    fmat = jnp.transpose(feats.reshape(16, B, N), (1, 0, 2))
    w1p = jnp.pad(W1, ((0, 0), (0, 16 - W1.shape[1])))
    out = pl.pallas_call(
        _mlp_kernel,
        grid=(B,),
        in_specs=[
            pl.BlockSpec((None, 16, N), lambda b: (b, 0, 0)),
            pl.BlockSpec((64, 16), lambda b: (0, 0)),
            pl.BlockSpec((64, 1), lambda b: (0, 0)),
            pl.BlockSpec((128, 64), lambda b: (0, 0)),
            pl.BlockSpec((128, 1), lambda b: (0, 0)),
            pl.BlockSpec((256, 128), lambda b: (0, 0)),
            pl.BlockSpec((256, 1), lambda b: (0, 0)),
        ],
        out_specs=pl.BlockSpec((None, 256, N), lambda b: (b, 0, 0)),
        out_shape=jax.ShapeDtypeStruct((B, 256, N), jnp.float32),
    )(fmat, w1p, b1[:, None], W2, b2[:, None], W3, b3[:, None])
    return out
